# baseline (speedup 1.0000x reference)
import functools

import jax
import jax.numpy as jnp
from jax import lax
from jax.experimental import pallas as pl
from jax.experimental.pallas import tpu as pltpu
from jax.experimental.pallas import tpu_sc as plsc

B, N, M, C1, C2 = 8, 4096, 1024, 256, 256
BN = B * N
TN = 512
TM = 1024

NC, NS = 2, 16
NW = NC * NS
Q = BN // NW
CH = 16
NCHUNK = Q // CH


def _knn_body(u_ref, k_ref, idx_ref, w_ref):
    b = pl.program_id(0)
    u = u_ref[0]
    k = k_ref[0]
    cross = lax.dot_general(u, k, (((1,), (1,)), ((), ())),
                            preferred_element_type=jnp.float32,
                            precision=lax.Precision.HIGHEST)
    un = jnp.sum(u * u, axis=1, keepdims=True)
    kn = jnp.sum(k * k, axis=1, keepdims=True).reshape(1, M)
    d2 = jnp.maximum(un + kn - 2.0 * cross, 0.0)

    iota = lax.broadcasted_iota(jnp.int32, (TN, M), 1)
    dists, idxs = [], []
    for _ in range(3):
        m = jnp.min(d2, axis=1, keepdims=True)
        i = jnp.min(jnp.where(d2 == m, iota, jnp.int32(M)),
                    axis=1, keepdims=True)
        d2 = jnp.where(iota == i, jnp.float32(jnp.inf), d2)
        dists.append(m)
        idxs.append(i)

    r = [1.0 / (d + 1e-10) for d in dists]
    norm = r[0] + r[1] + r[2]
    w_ref[0] = jnp.concatenate([x / norm for x in r], axis=1)
    idx_ref[0] = jnp.concatenate(idxs, axis=1) + b * M


def _knn(unknown, known):
    return pl.pallas_call(
        _knn_body,
        grid=(B, N // TN),
        in_specs=[
            pl.BlockSpec((1, TN, 3), lambda b, i: (b, i, 0)),
            pl.BlockSpec((1, M, 3), lambda b, i: (b, 0, 0)),
        ],
        out_specs=[
            pl.BlockSpec((1, TN, 3), lambda b, i: (b, i, 0)),
            pl.BlockSpec((1, TN, 3), lambda b, i: (b, i, 0)),
        ],
        out_shape=[
            jax.ShapeDtypeStruct((B, N, 3), jnp.int32),
            jax.ShapeDtypeStruct((B, N, 3), jnp.float32),
        ],
    )(unknown, known)


def _sc_interp_body(kf_hbm, idx_hbm, w_hbm, out_hbm, idx_v, w_v, rows_v,
                    out_v, sem):
    wid = lax.axis_index("s") * NC + lax.axis_index("c")

    def chunk(t, _):
        base_r = wid * Q + t * CH
        base_i = base_r * 3
        pltpu.sync_copy(idx_hbm.at[pl.ds(base_i, CH * 3)], idx_v)
        pltpu.sync_copy(w_hbm.at[pl.ds(base_i, CH * 3)], w_v)
        pltpu.async_copy(kf_hbm.at[idx_v], rows_v, sem).wait()
        for rr in range(CH):
            for g in range(C2 // 16):
                s = pl.ds(g * 16, 16)
                acc = w_v[3 * rr, :] * rows_v[3 * rr, s]
                acc = acc + w_v[3 * rr + 1, :] * rows_v[3 * rr + 1, s]
                acc = acc + w_v[3 * rr + 2, :] * rows_v[3 * rr + 2, s]
                out_v[rr, s] = acc
        pltpu.sync_copy(out_v, out_hbm.at[pl.ds(base_r, CH)])
        return _

    lax.fori_loop(0, NCHUNK, chunk, None)


def _sc_interp(kf_flat, idx_flat, wexp):
    run = pl.kernel(
        _sc_interp_body,
        mesh=plsc.VectorSubcoreMesh(core_axis_name="c", subcore_axis_name="s"),
        out_type=jax.ShapeDtypeStruct((BN, C2), jnp.float32),
        scratch_types=[
            pltpu.VMEM((CH * 3,), jnp.int32),
            pltpu.VMEM((CH * 3, 16), jnp.float32),
            pltpu.VMEM((CH * 3, C2), jnp.float32),
            pltpu.VMEM((CH, C2), jnp.float32),
            pltpu.SemaphoreType.DMA,
        ],
    )
    return run(kf_flat, idx_flat, wexp)


def _mm_stats_body(x1_ref, x2_ref, wa_ref, wb_ref, b_ref, sc_ref, sh_ref,
                   h_ref, s_ref, q_ref, *, relu_in):
    x1 = x1_ref[...]
    if relu_in:
        x1 = jnp.maximum(x1 * sc_ref[...] + sh_ref[...], 0.0)
    h = jnp.dot(x1, wa_ref[...], preferred_element_type=jnp.float32,
                precision=lax.Precision.HIGHEST)
    if x2_ref is not None:
        h = h + jnp.dot(x2_ref[...], wb_ref[...],
                        preferred_element_type=jnp.float32,
                        precision=lax.Precision.HIGHEST)
    h = h + b_ref[...]
    h_ref[...] = h

    @pl.when(pl.program_id(0) == 0)
    def _():
        s_ref[...] = jnp.zeros_like(s_ref)
        q_ref[...] = jnp.zeros_like(q_ref)

    s_ref[...] += jnp.sum(h, axis=0, keepdims=True)
    q_ref[...] += jnp.sum(h * h, axis=0, keepdims=True)


def _pass_a(interp, unk, w1at, w1bt, b1):
    body = functools.partial(_mm_stats_body, relu_in=False)

    def wrapped(x1, x2, wa, wb, bb, h, s, q):
        body(x1, x2, wa, wb, bb, None, None, h, s, q)

    row = pl.BlockSpec((TM, C2), lambda i: (i, 0))
    full = pl.BlockSpec((C2, C2), lambda i: (0, 0))
    vec = pl.BlockSpec((1, C2), lambda i: (0, 0))
    return pl.pallas_call(
        wrapped,
        grid=(BN // TM,),
        in_specs=[row, row, full, full, vec],
        out_specs=[row, vec, vec],
        out_shape=[
            jax.ShapeDtypeStruct((BN, C2), jnp.float32),
            jax.ShapeDtypeStruct((1, C2), jnp.float32),
            jax.ShapeDtypeStruct((1, C2), jnp.float32),
        ],
    )(interp, unk, w1at, w1bt, b1)


def _pass_b(h1, w2t, b2, scale1, shift1):
    def wrapped(x1, wa, bb, sc, sh, h, s, q):
        _mm_stats_body(x1, None, wa, None, bb, sc, sh, h, s, q, relu_in=True)

    row = pl.BlockSpec((TM, C2), lambda i: (i, 0))
    full = pl.BlockSpec((C2, C2), lambda i: (0, 0))
    vec = pl.BlockSpec((1, C2), lambda i: (0, 0))
    return pl.pallas_call(
        wrapped,
        grid=(BN // TM,),
        in_specs=[row, full, vec, vec, vec],
        out_specs=[row, vec, vec],
        out_shape=[
            jax.ShapeDtypeStruct((BN, C2), jnp.float32),
            jax.ShapeDtypeStruct((1, C2), jnp.float32),
            jax.ShapeDtypeStruct((1, C2), jnp.float32),
        ],
    )(h1, w2t, b2, scale1, shift1)


def _pass_c_body(h_ref, sc_ref, sh_ref, o_ref):
    o_ref[...] = jnp.maximum(h_ref[...] * sc_ref[...] + sh_ref[...], 0.0)


def _pass_c(h2, scale2, shift2):
    row = pl.BlockSpec((TM, C2), lambda i: (i, 0))
    vec = pl.BlockSpec((1, C2), lambda i: (0, 0))
    return pl.pallas_call(
        _pass_c_body,
        grid=(BN // TM,),
        in_specs=[row, vec, vec],
        out_specs=row,
        out_shape=jax.ShapeDtypeStruct((BN, C2), jnp.float32),
    )(h2, scale2, shift2)


def _affine(s, q, g, beta, eps=1e-5):
    mu = s / BN
    var = q / BN - mu * mu
    scale = g.reshape(1, -1) * lax.rsqrt(var + eps)
    shift = beta.reshape(1, -1) - mu * scale
    return scale, shift


def kernel(unknown, known, unknow_feats, known_feats, W1, b1, g1, beta1,
           W2, b2, g2, beta2):
    idx, w = _knn(unknown, known)

    idx_flat = idx.reshape(BN * 3)
    wexp = jnp.broadcast_to(w.reshape(BN * 3, 1), (BN * 3, 16))
    kf_flat = known_feats.reshape(B * M, C2)
    interp = (kf_flat[idx_flat] * wexp[:, :1]).reshape(BN, 3, C2).sum(axis=1)

    unk = unknow_feats.reshape(BN, C1)
    w1at = W1[:, :C2].T
    w1bt = W1[:, C2:].T
    h1, s1, q1 = _pass_a(interp, unk, w1at, w1bt, b1.reshape(1, C2))
    scale1, shift1 = _affine(s1, q1, g1, beta1)
    h2, s2, q2 = _pass_b(h1, W2.T, b2.reshape(1, C2), scale1, shift1)
    scale2, shift2 = _affine(s2, q2, g2, beta2)
    out = _pass_c(h2, scale2, shift2)
    return out.reshape(B, N, C2)


# speedup vs baseline: 1.2338x; 1.2338x over previous
import functools

import jax
import jax.numpy as jnp
from jax import lax
from jax.experimental import pallas as pl
from jax.experimental.pallas import tpu as pltpu
from jax.experimental.pallas import tpu_sc as plsc

B, N, M, C1, C2 = 8, 4096, 1024, 256, 256
BN = B * N
TN = 512
TM = 1024

NC, NS = 2, 16
NW = NC * NS
Q = BN // NW
CH = 16
NCHUNK = Q // CH


def _knn_body(u_ref, k_ref, idx_ref, w_ref):
    b = pl.program_id(0)
    u = u_ref[0]
    k = k_ref[0]
    cross = lax.dot_general(u, k, (((1,), (1,)), ((), ())),
                            preferred_element_type=jnp.float32,
                            precision=lax.Precision.HIGHEST)
    un = jnp.sum(u * u, axis=1, keepdims=True)
    kn = jnp.sum(k * k, axis=1, keepdims=True).reshape(1, M)
    d2 = jnp.maximum(un + kn - 2.0 * cross, 0.0)

    iota = lax.broadcasted_iota(jnp.int32, (TN, M), 1)
    dists, idxs = [], []
    for _ in range(3):
        m = jnp.min(d2, axis=1, keepdims=True)
        i = jnp.min(jnp.where(d2 == m, iota, jnp.int32(M)),
                    axis=1, keepdims=True)
        d2 = jnp.where(iota == i, jnp.float32(jnp.inf), d2)
        dists.append(m)
        idxs.append(i)

    r = [1.0 / (d + 1e-10) for d in dists]
    norm = r[0] + r[1] + r[2]
    w_ref[0] = jnp.concatenate([x / norm for x in r], axis=1)
    idx_ref[0] = jnp.concatenate(idxs, axis=1) + b * M


def _knn(unknown, known):
    return pl.pallas_call(
        _knn_body,
        grid=(B, N // TN),
        in_specs=[
            pl.BlockSpec((1, TN, 3), lambda b, i: (b, i, 0)),
            pl.BlockSpec((1, M, 3), lambda b, i: (b, 0, 0)),
        ],
        out_specs=[
            pl.BlockSpec((1, TN, 3), lambda b, i: (b, i, 0)),
            pl.BlockSpec((1, TN, 3), lambda b, i: (b, i, 0)),
        ],
        out_shape=[
            jax.ShapeDtypeStruct((B, N, 3), jnp.int32),
            jax.ShapeDtypeStruct((B, N, 3), jnp.float32),
        ],
    )(unknown, known)


def _sc_interp_body(kf_hbm, idx_hbm, w_hbm, out_hbm, idx_v, w_v, rows_v,
                    out_v, sem):
    wid = lax.axis_index("s") * NC + lax.axis_index("c")

    def chunk(t, _):
        base_r = wid * Q + t * CH
        base_i = base_r * 3
        pltpu.sync_copy(idx_hbm.at[pl.ds(base_i, CH * 3)], idx_v)
        pltpu.sync_copy(w_hbm.at[pl.ds(base_i, CH * 3)], w_v)
        pltpu.async_copy(kf_hbm.at[idx_v], rows_v, sem).wait()
        for rr in range(CH):
            for g in range(C2 // 16):
                s = pl.ds(g * 16, 16)
                acc = w_v[3 * rr, :] * rows_v[3 * rr, s]
                acc = acc + w_v[3 * rr + 1, :] * rows_v[3 * rr + 1, s]
                acc = acc + w_v[3 * rr + 2, :] * rows_v[3 * rr + 2, s]
                out_v[rr, s] = acc
        pltpu.sync_copy(out_v, out_hbm.at[pl.ds(base_r, CH)])
        return _

    lax.fori_loop(0, NCHUNK, chunk, None)


def _sc_interp(kf_flat, idx_flat, wexp):
    run = pl.kernel(
        _sc_interp_body,
        mesh=plsc.VectorSubcoreMesh(core_axis_name="c", subcore_axis_name="s"),
        out_type=jax.ShapeDtypeStruct((BN, C2), jnp.float32),
        scratch_types=[
            pltpu.VMEM((CH * 3,), jnp.int32),
            pltpu.VMEM((CH * 3, 16), jnp.float32),
            pltpu.VMEM((CH * 3, C2), jnp.float32),
            pltpu.VMEM((CH, C2), jnp.float32),
            pltpu.SemaphoreType.DMA,
        ],
    )
    return run(kf_flat, idx_flat, wexp)


def _mm_stats_body(x1_ref, x2_ref, wa_ref, wb_ref, b_ref, sc_ref, sh_ref,
                   h_ref, s_ref, q_ref, *, relu_in):
    x1 = x1_ref[...]
    if relu_in:
        x1 = jnp.maximum(x1 * sc_ref[...] + sh_ref[...], 0.0)
    h = jnp.dot(x1, wa_ref[...], preferred_element_type=jnp.float32)
    if x2_ref is not None:
        h = h + jnp.dot(x2_ref[...], wb_ref[...],
                        preferred_element_type=jnp.float32)
    h = h + b_ref[...]
    h_ref[...] = h

    @pl.when(pl.program_id(0) == 0)
    def _():
        s_ref[...] = jnp.zeros_like(s_ref)
        q_ref[...] = jnp.zeros_like(q_ref)

    s_ref[...] += jnp.sum(h, axis=0, keepdims=True)
    q_ref[...] += jnp.sum(h * h, axis=0, keepdims=True)


def _pass_a(interp, unk, w1at, w1bt, b1):
    body = functools.partial(_mm_stats_body, relu_in=False)

    def wrapped(x1, x2, wa, wb, bb, h, s, q):
        body(x1, x2, wa, wb, bb, None, None, h, s, q)

    row = pl.BlockSpec((TM, C2), lambda i: (i, 0))
    full = pl.BlockSpec((C2, C2), lambda i: (0, 0))
    vec = pl.BlockSpec((1, C2), lambda i: (0, 0))
    return pl.pallas_call(
        wrapped,
        grid=(BN // TM,),
        in_specs=[row, row, full, full, vec],
        out_specs=[row, vec, vec],
        out_shape=[
            jax.ShapeDtypeStruct((BN, C2), jnp.float32),
            jax.ShapeDtypeStruct((1, C2), jnp.float32),
            jax.ShapeDtypeStruct((1, C2), jnp.float32),
        ],
    )(interp, unk, w1at, w1bt, b1)


def _pass_b(h1, w2t, b2, scale1, shift1):
    def wrapped(x1, wa, bb, sc, sh, h, s, q):
        _mm_stats_body(x1, None, wa, None, bb, sc, sh, h, s, q, relu_in=True)

    row = pl.BlockSpec((TM, C2), lambda i: (i, 0))
    full = pl.BlockSpec((C2, C2), lambda i: (0, 0))
    vec = pl.BlockSpec((1, C2), lambda i: (0, 0))
    return pl.pallas_call(
        wrapped,
        grid=(BN // TM,),
        in_specs=[row, full, vec, vec, vec],
        out_specs=[row, vec, vec],
        out_shape=[
            jax.ShapeDtypeStruct((BN, C2), jnp.float32),
            jax.ShapeDtypeStruct((1, C2), jnp.float32),
            jax.ShapeDtypeStruct((1, C2), jnp.float32),
        ],
    )(h1, w2t, b2, scale1, shift1)


def _pass_c_body(h_ref, sc_ref, sh_ref, o_ref):
    o_ref[...] = jnp.maximum(h_ref[...] * sc_ref[...] + sh_ref[...], 0.0)


def _pass_c(h2, scale2, shift2):
    row = pl.BlockSpec((TM, C2), lambda i: (i, 0))
    vec = pl.BlockSpec((1, C2), lambda i: (0, 0))
    return pl.pallas_call(
        _pass_c_body,
        grid=(BN // TM,),
        in_specs=[row, vec, vec],
        out_specs=row,
        out_shape=jax.ShapeDtypeStruct((BN, C2), jnp.float32),
    )(h2, scale2, shift2)


def _affine(s, q, g, beta, eps=1e-5):
    mu = s / BN
    var = q / BN - mu * mu
    scale = g.reshape(1, -1) * lax.rsqrt(var + eps)
    shift = beta.reshape(1, -1) - mu * scale
    return scale, shift


def kernel(unknown, known, unknow_feats, known_feats, W1, b1, g1, beta1,
           W2, b2, g2, beta2):
    idx, w = _knn(unknown, known)

    idx_flat = idx.reshape(BN * 3)
    wexp = jnp.broadcast_to(w.reshape(BN * 3, 1), (BN * 3, 16))
    kf_flat = known_feats.reshape(B * M, C2)
    interp = _sc_interp(kf_flat, idx_flat, wexp)

    unk = unknow_feats.reshape(BN, C1)
    w1at = W1[:, :C2].T
    w1bt = W1[:, C2:].T
    h1, s1, q1 = _pass_a(interp, unk, w1at, w1bt, b1.reshape(1, C2))
    scale1, shift1 = _affine(s1, q1, g1, beta1)
    h2, s2, q2 = _pass_b(h1, W2.T, b2.reshape(1, C2), scale1, shift1)
    scale2, shift2 = _affine(s2, q2, g2, beta2)
    out = _pass_c(h2, scale2, shift2)
    return out.reshape(B, N, C2)


# speedup vs baseline: 2.0869x; 1.6914x over previous
import functools

import jax
import jax.numpy as jnp
from jax import lax
from jax.experimental import pallas as pl
from jax.experimental.pallas import tpu as pltpu
from jax.experimental.pallas import tpu_sc as plsc

B, N, M, C1, C2 = 8, 4096, 1024, 256, 256
BN = B * N
TN = 512
TM = 1024

NC, NS = 2, 16
NW = NC * NS
Q = BN // NW
CH = 16
NCHUNK = Q // CH


def _knn_body(u_ref, k_ref, idx_ref, w_ref):
    b = pl.program_id(0)
    u = u_ref[0]
    k = k_ref[0]
    cross = lax.dot_general(u, k, (((1,), (1,)), ((), ())),
                            preferred_element_type=jnp.float32,
                            precision=lax.Precision.HIGHEST)
    un = jnp.sum(u * u, axis=1, keepdims=True)
    kn = jnp.sum(k * k, axis=1, keepdims=True).reshape(1, M)
    d2 = jnp.maximum(un + kn - 2.0 * cross, 0.0)

    iota = lax.broadcasted_iota(jnp.int32, (TN, M), 1)
    dists, idxs = [], []
    for _ in range(3):
        m = jnp.min(d2, axis=1, keepdims=True)
        i = jnp.min(jnp.where(d2 == m, iota, jnp.int32(M)),
                    axis=1, keepdims=True)
        d2 = jnp.where(iota == i, jnp.float32(jnp.inf), d2)
        dists.append(m)
        idxs.append(i)

    r = [1.0 / (d + 1e-10) for d in dists]
    norm = r[0] + r[1] + r[2]
    w_ref[0] = jnp.concatenate([x / norm for x in r], axis=1)
    idx_ref[0] = jnp.concatenate(idxs, axis=1) + b * M


def _knn(unknown, known):
    return pl.pallas_call(
        _knn_body,
        grid=(B, N // TN),
        in_specs=[
            pl.BlockSpec((1, TN, 3), lambda b, i: (b, i, 0)),
            pl.BlockSpec((1, M, 3), lambda b, i: (b, 0, 0)),
        ],
        out_specs=[
            pl.BlockSpec((1, TN, 3), lambda b, i: (b, i, 0)),
            pl.BlockSpec((1, TN, 3), lambda b, i: (b, i, 0)),
        ],
        out_shape=[
            jax.ShapeDtypeStruct((B, N, 3), jnp.int32),
            jax.ShapeDtypeStruct((B, N, 3), jnp.float32),
        ],
    )(unknown, known)


def _sc_interp_body(kf_hbm, idx_hbm, w_hbm, out_hbm, idx_v, w_v, rv0, rv1,
                    ov0, ov1, sg0, sg1, so0, so1):
    wid = lax.axis_index("s") * NC + lax.axis_index("c")
    base_r0 = pl.multiple_of(wid * Q, Q)
    base_i0 = pl.multiple_of(base_r0 * 3, Q * 3)

    def gather(t, rv, sem):
        return pltpu.async_copy(
            kf_hbm.at[idx_v.at[pl.ds(pl.multiple_of(t * (CH * 3), CH * 3),
                                     CH * 3)]], rv, sem)

    def wait_gather(rv, sem):
        pltpu.make_async_copy(kf_hbm.at[pl.ds(0, CH * 3)], rv, sem).wait()

    def out_copy(t, ov, sem):
        return pltpu.async_copy(
            ov, out_hbm.at[pl.ds(base_r0 + pl.multiple_of(t * CH, CH), CH)],
            sem)

    def wait_out(ov, sem):
        pltpu.make_async_copy(ov, out_hbm.at[pl.ds(base_r0, CH)], sem).wait()

    def compute(t, rv, ov):
        wv = [w_v[pl.ds(pl.multiple_of(t * (CH * 3), CH * 3) + 16 * kk, 16)]
              for kk in range(3)]
        for rr in range(CH):
            ws = []
            for j in range(3):
                e = 3 * rr + j
                ws.append(jnp.full((16,), wv[e // 16][e % 16], jnp.float32))
            for g in range(C2 // 16):
                s = pl.ds(g * 16, 16)
                acc = ws[0] * rv[3 * rr, s]
                acc = acc + ws[1] * rv[3 * rr + 1, s]
                acc = acc + ws[2] * rv[3 * rr + 2, s]
                ov[rr, s] = acc

    pltpu.sync_copy(idx_hbm.at[pl.ds(base_i0, Q * 3)], idx_v)
    pltpu.sync_copy(w_hbm.at[pl.ds(base_i0, Q * 3)], w_v)

    gather(0, rv0, sg0).wait()
    gather(1, rv1, sg1)
    compute(0, rv0, ov0)
    out_copy(0, ov0, so0)
    gather(2, rv0, sg0)
    wait_gather(rv1, sg1)
    compute(1, rv1, ov1)
    out_copy(1, ov1, so1)

    def pair(p, _):
        t0 = 2 * p
        wait_gather(rv0, sg0)
        gather(t0 + 1, rv1, sg1)
        wait_out(ov0, so0)
        compute(t0, rv0, ov0)
        out_copy(t0, ov0, so0)
        wait_gather(rv1, sg1)
        g_next = jnp.minimum(t0 + 2, NCHUNK - 2)
        gather(g_next, rv0, sg0)
        wait_out(ov1, so1)
        compute(t0 + 1, rv1, ov1)
        out_copy(t0 + 1, ov1, so1)
        return _

    lax.fori_loop(1, NCHUNK // 2, pair, None)
    wait_gather(rv0, sg0)
    wait_out(ov0, so0)
    wait_out(ov1, so1)


def _sc_interp(kf_flat, idx_flat, wexp):
    run = pl.kernel(
        _sc_interp_body,
        mesh=plsc.VectorSubcoreMesh(core_axis_name="c", subcore_axis_name="s"),
        out_type=jax.ShapeDtypeStruct((BN, C2), jnp.float32),
        scratch_types=[
            pltpu.VMEM((Q * 3,), jnp.int32),
            pltpu.VMEM((Q * 3,), jnp.float32),
            pltpu.VMEM((CH * 3, C2), jnp.float32),
            pltpu.VMEM((CH * 3, C2), jnp.float32),
            pltpu.VMEM((CH, C2), jnp.float32),
            pltpu.VMEM((CH, C2), jnp.float32),
            pltpu.SemaphoreType.DMA,
            pltpu.SemaphoreType.DMA,
            pltpu.SemaphoreType.DMA,
            pltpu.SemaphoreType.DMA,
        ],
    )
    return run(kf_flat, idx_flat, wexp)


def _mm_stats_body(x1_ref, x2_ref, wa_ref, wb_ref, b_ref, sc_ref, sh_ref,
                   h_ref, s_ref, q_ref, *, relu_in):
    x1 = x1_ref[...]
    if relu_in:
        x1 = jnp.maximum(x1 * sc_ref[...] + sh_ref[...], 0.0)
    h = jnp.dot(x1, wa_ref[...], preferred_element_type=jnp.float32)
    if x2_ref is not None:
        h = h + jnp.dot(x2_ref[...], wb_ref[...],
                        preferred_element_type=jnp.float32)
    h = h + b_ref[...]
    h_ref[...] = h

    @pl.when(pl.program_id(0) == 0)
    def _():
        s_ref[...] = jnp.zeros_like(s_ref)
        q_ref[...] = jnp.zeros_like(q_ref)

    s_ref[...] += jnp.sum(h, axis=0, keepdims=True)
    q_ref[...] += jnp.sum(h * h, axis=0, keepdims=True)


def _pass_a(interp, unk, w1at, w1bt, b1):
    body = functools.partial(_mm_stats_body, relu_in=False)

    def wrapped(x1, x2, wa, wb, bb, h, s, q):
        body(x1, x2, wa, wb, bb, None, None, h, s, q)

    row = pl.BlockSpec((TM, C2), lambda i: (i, 0))
    full = pl.BlockSpec((C2, C2), lambda i: (0, 0))
    vec = pl.BlockSpec((1, C2), lambda i: (0, 0))
    return pl.pallas_call(
        wrapped,
        grid=(BN // TM,),
        in_specs=[row, row, full, full, vec],
        out_specs=[row, vec, vec],
        out_shape=[
            jax.ShapeDtypeStruct((BN, C2), jnp.float32),
            jax.ShapeDtypeStruct((1, C2), jnp.float32),
            jax.ShapeDtypeStruct((1, C2), jnp.float32),
        ],
    )(interp, unk, w1at, w1bt, b1)


def _pass_b(h1, w2t, b2, scale1, shift1):
    def wrapped(x1, wa, bb, sc, sh, h, s, q):
        _mm_stats_body(x1, None, wa, None, bb, sc, sh, h, s, q, relu_in=True)

    row = pl.BlockSpec((TM, C2), lambda i: (i, 0))
    full = pl.BlockSpec((C2, C2), lambda i: (0, 0))
    vec = pl.BlockSpec((1, C2), lambda i: (0, 0))
    return pl.pallas_call(
        wrapped,
        grid=(BN // TM,),
        in_specs=[row, full, vec, vec, vec],
        out_specs=[row, vec, vec],
        out_shape=[
            jax.ShapeDtypeStruct((BN, C2), jnp.float32),
            jax.ShapeDtypeStruct((1, C2), jnp.float32),
            jax.ShapeDtypeStruct((1, C2), jnp.float32),
        ],
    )(h1, w2t, b2, scale1, shift1)


def _pass_c_body(h_ref, sc_ref, sh_ref, o_ref):
    o_ref[...] = jnp.maximum(h_ref[...] * sc_ref[...] + sh_ref[...], 0.0)


def _pass_c(h2, scale2, shift2):
    row = pl.BlockSpec((TM, C2), lambda i: (i, 0))
    vec = pl.BlockSpec((1, C2), lambda i: (0, 0))
    return pl.pallas_call(
        _pass_c_body,
        grid=(BN // TM,),
        in_specs=[row, vec, vec],
        out_specs=row,
        out_shape=jax.ShapeDtypeStruct((BN, C2), jnp.float32),
    )(h2, scale2, shift2)


def _affine(s, q, g, beta, eps=1e-5):
    mu = s / BN
    var = q / BN - mu * mu
    scale = g.reshape(1, -1) * lax.rsqrt(var + eps)
    shift = beta.reshape(1, -1) - mu * scale
    return scale, shift


def kernel(unknown, known, unknow_feats, known_feats, W1, b1, g1, beta1,
           W2, b2, g2, beta2):
    idx, w = _knn(unknown, known)

    idx_flat = idx.reshape(BN * 3)
    kf_flat = known_feats.reshape(B * M, C2)
    interp = _sc_interp(kf_flat, idx_flat, w.reshape(BN * 3))

    unk = unknow_feats.reshape(BN, C1)
    w1at = W1[:, :C2].T
    w1bt = W1[:, C2:].T
    h1, s1, q1 = _pass_a(interp, unk, w1at, w1bt, b1.reshape(1, C2))
    scale1, shift1 = _affine(s1, q1, g1, beta1)
    h2, s2, q2 = _pass_b(h1, W2.T, b2.reshape(1, C2), scale1, shift1)
    scale2, shift2 = _affine(s2, q2, g2, beta2)
    out = _pass_c(h2, scale2, shift2)
    return out.reshape(B, N, C2)


# speedup vs baseline: 2.0876x; 1.0003x over previous
import functools

import jax
import jax.numpy as jnp
from jax import lax
from jax.experimental import pallas as pl
from jax.experimental.pallas import tpu as pltpu
from jax.experimental.pallas import tpu_sc as plsc

B, N, M, C1, C2 = 8, 4096, 1024, 256, 256
BN = B * N
TN = 1024
TM = 1024

NC, NS = 2, 16
NW = NC * NS
Q = BN // NW
CH = 16
NCHUNK = Q // CH


def _knn_body(u_ref, k_ref, idx_ref, w_ref):
    b = pl.program_id(0)
    u4 = u_ref[0]
    k4 = k_ref[0]
    sel = lax.dot_general(u4, k4, (((1,), (1,)), ((), ())),
                          preferred_element_type=jnp.float32,
                          precision=lax.Precision.HIGHEST)
    u3 = u4[:, :3]
    un = jnp.sum(u3 * u3, axis=1, keepdims=True)

    iota = lax.broadcasted_iota(jnp.int32, (TN, M), 1)
    dists, idxs = [], []
    for _ in range(3):
        m = jnp.min(sel, axis=1, keepdims=True)
        i = jnp.min(jnp.where(sel == m, iota, jnp.int32(M)),
                    axis=1, keepdims=True)
        sel = jnp.where(iota == i, jnp.float32(jnp.inf), sel)
        dists.append(jnp.maximum(m + un, 0.0))
        idxs.append(i)

    r = [1.0 / (d + 1e-10) for d in dists]
    norm = r[0] + r[1] + r[2]
    w_ref[0] = jnp.concatenate([x / norm for x in r], axis=1)
    idx_ref[0] = jnp.concatenate(idxs, axis=1) + b * M


def _knn(u4, k4):
    return pl.pallas_call(
        _knn_body,
        grid=(B, N // TN),
        in_specs=[
            pl.BlockSpec((1, TN, 4), lambda b, i: (b, i, 0)),
            pl.BlockSpec((1, M, 4), lambda b, i: (b, 0, 0)),
        ],
        out_specs=[
            pl.BlockSpec((1, TN, 3), lambda b, i: (b, i, 0)),
            pl.BlockSpec((1, TN, 3), lambda b, i: (b, i, 0)),
        ],
        out_shape=[
            jax.ShapeDtypeStruct((B, N, 3), jnp.int32),
            jax.ShapeDtypeStruct((B, N, 3), jnp.float32),
        ],
    )(u4, k4)


def _sc_interp_body(kf_hbm, idx_hbm, w_hbm, out_hbm, idx_v, w_v, rv0, rv1,
                    ov0, ov1, sg0, sg1, so0, so1):
    wid = lax.axis_index("s") * NC + lax.axis_index("c")
    base_r0 = pl.multiple_of(wid * Q, Q)
    base_i0 = pl.multiple_of(base_r0 * 3, Q * 3)

    def gather(t, rv, sem):
        return pltpu.async_copy(
            kf_hbm.at[idx_v.at[pl.ds(pl.multiple_of(t * (CH * 3), CH * 3),
                                     CH * 3)]], rv, sem)

    def wait_gather(rv, sem):
        pltpu.make_async_copy(kf_hbm.at[pl.ds(0, CH * 3)], rv, sem).wait()

    def out_copy(t, ov, sem):
        return pltpu.async_copy(
            ov, out_hbm.at[pl.ds(base_r0 + pl.multiple_of(t * CH, CH), CH)],
            sem)

    def wait_out(ov, sem):
        pltpu.make_async_copy(ov, out_hbm.at[pl.ds(base_r0, CH)], sem).wait()

    def compute(t, rv, ov):
        wv = [w_v[pl.ds(pl.multiple_of(t * (CH * 3), CH * 3) + 16 * kk, 16)]
              for kk in range(3)]
        for rr in range(CH):
            ws = []
            for j in range(3):
                e = 3 * rr + j
                ws.append(jnp.full((16,), wv[e // 16][e % 16], jnp.float32))
            for g in range(C2 // 16):
                s = pl.ds(g * 16, 16)
                acc = ws[0] * rv[3 * rr, s]
                acc = acc + ws[1] * rv[3 * rr + 1, s]
                acc = acc + ws[2] * rv[3 * rr + 2, s]
                ov[rr, s] = acc

    pltpu.sync_copy(idx_hbm.at[pl.ds(base_i0, Q * 3)], idx_v)
    pltpu.sync_copy(w_hbm.at[pl.ds(base_i0, Q * 3)], w_v)

    gather(0, rv0, sg0).wait()
    gather(1, rv1, sg1)
    compute(0, rv0, ov0)
    out_copy(0, ov0, so0)
    gather(2, rv0, sg0)
    wait_gather(rv1, sg1)
    compute(1, rv1, ov1)
    out_copy(1, ov1, so1)

    def pair(p, _):
        t0 = 2 * p
        wait_gather(rv0, sg0)
        gather(t0 + 1, rv1, sg1)
        wait_out(ov0, so0)
        compute(t0, rv0, ov0)
        out_copy(t0, ov0, so0)
        wait_gather(rv1, sg1)
        g_next = jnp.minimum(t0 + 2, NCHUNK - 2)
        gather(g_next, rv0, sg0)
        wait_out(ov1, so1)
        compute(t0 + 1, rv1, ov1)
        out_copy(t0 + 1, ov1, so1)
        return _

    lax.fori_loop(1, NCHUNK // 2, pair, None)
    wait_gather(rv0, sg0)
    wait_out(ov0, so0)
    wait_out(ov1, so1)


def _sc_interp(kf_flat, idx_flat, wexp):
    run = pl.kernel(
        _sc_interp_body,
        mesh=plsc.VectorSubcoreMesh(core_axis_name="c", subcore_axis_name="s"),
        out_type=jax.ShapeDtypeStruct((BN, C2), jnp.float32),
        scratch_types=[
            pltpu.VMEM((Q * 3,), jnp.int32),
            pltpu.VMEM((Q * 3,), jnp.float32),
            pltpu.VMEM((CH * 3, C2), jnp.float32),
            pltpu.VMEM((CH * 3, C2), jnp.float32),
            pltpu.VMEM((CH, C2), jnp.float32),
            pltpu.VMEM((CH, C2), jnp.float32),
            pltpu.SemaphoreType.DMA,
            pltpu.SemaphoreType.DMA,
            pltpu.SemaphoreType.DMA,
            pltpu.SemaphoreType.DMA,
        ],
    )
    return run(kf_flat, idx_flat, wexp)


def _mm_stats_body(x1_ref, x2_ref, wa_ref, wb_ref, b_ref, sc_ref, sh_ref,
                   h_ref, s_ref, q_ref, *, relu_in):
    x1 = x1_ref[...]
    if relu_in:
        x1 = jnp.maximum(x1 * sc_ref[...] + sh_ref[...], 0.0)
    h = jnp.dot(x1, wa_ref[...], preferred_element_type=jnp.float32)
    if x2_ref is not None:
        h = h + jnp.dot(x2_ref[...], wb_ref[...],
                        preferred_element_type=jnp.float32)
    h = h + b_ref[...]
    h_ref[...] = h

    @pl.when(pl.program_id(0) == 0)
    def _():
        s_ref[...] = jnp.zeros_like(s_ref)
        q_ref[...] = jnp.zeros_like(q_ref)

    s_ref[...] += jnp.sum(h, axis=0, keepdims=True)
    q_ref[...] += jnp.sum(h * h, axis=0, keepdims=True)


def _pass_a(interp, unk, w1at, w1bt, b1):
    body = functools.partial(_mm_stats_body, relu_in=False)

    def wrapped(x1, x2, wa, wb, bb, h, s, q):
        body(x1, x2, wa, wb, bb, None, None, h, s, q)

    row = pl.BlockSpec((TM, C2), lambda i: (i, 0))
    full = pl.BlockSpec((C2, C2), lambda i: (0, 0))
    vec = pl.BlockSpec((1, C2), lambda i: (0, 0))
    return pl.pallas_call(
        wrapped,
        grid=(BN // TM,),
        in_specs=[row, row, full, full, vec],
        out_specs=[row, vec, vec],
        out_shape=[
            jax.ShapeDtypeStruct((BN, C2), jnp.float32),
            jax.ShapeDtypeStruct((1, C2), jnp.float32),
            jax.ShapeDtypeStruct((1, C2), jnp.float32),
        ],
    )(interp, unk, w1at, w1bt, b1)


def _pass_b(h1, w2t, b2, scale1, shift1):
    def wrapped(x1, wa, bb, sc, sh, h, s, q):
        _mm_stats_body(x1, None, wa, None, bb, sc, sh, h, s, q, relu_in=True)

    row = pl.BlockSpec((TM, C2), lambda i: (i, 0))
    full = pl.BlockSpec((C2, C2), lambda i: (0, 0))
    vec = pl.BlockSpec((1, C2), lambda i: (0, 0))
    return pl.pallas_call(
        wrapped,
        grid=(BN // TM,),
        in_specs=[row, full, vec, vec, vec],
        out_specs=[row, vec, vec],
        out_shape=[
            jax.ShapeDtypeStruct((BN, C2), jnp.float32),
            jax.ShapeDtypeStruct((1, C2), jnp.float32),
            jax.ShapeDtypeStruct((1, C2), jnp.float32),
        ],
    )(h1, w2t, b2, scale1, shift1)


def _pass_c_body(h_ref, sc_ref, sh_ref, o_ref):
    o_ref[...] = jnp.maximum(h_ref[...] * sc_ref[...] + sh_ref[...], 0.0)


def _pass_c(h2, scale2, shift2):
    row = pl.BlockSpec((TM, C2), lambda i: (i, 0))
    vec = pl.BlockSpec((1, C2), lambda i: (0, 0))
    return pl.pallas_call(
        _pass_c_body,
        grid=(BN // TM,),
        in_specs=[row, vec, vec],
        out_specs=row,
        out_shape=jax.ShapeDtypeStruct((BN, C2), jnp.float32),
    )(h2, scale2, shift2)


def _affine(s, q, g, beta, eps=1e-5):
    mu = s / BN
    var = q / BN - mu * mu
    scale = g.reshape(1, -1) * lax.rsqrt(var + eps)
    shift = beta.reshape(1, -1) - mu * scale
    return scale, shift


def kernel(unknown, known, unknow_feats, known_feats, W1, b1, g1, beta1,
           W2, b2, g2, beta2):
    ones = jnp.ones((B, N, 1), jnp.float32)
    u4 = jnp.concatenate([unknown, ones], axis=2)
    kn = jnp.sum(known * known, axis=2, keepdims=True)
    k4 = jnp.concatenate([-2.0 * known, kn], axis=2)
    idx, w = _knn(u4, k4)

    idx_flat = idx.reshape(BN * 3)
    kf_flat = known_feats.reshape(B * M, C2)
    interp = _sc_interp(kf_flat, idx_flat, w.reshape(BN * 3))

    unk = unknow_feats.reshape(BN, C1)
    w1at = W1[:, :C2].T
    w1bt = W1[:, C2:].T
    h1, s1, q1 = _pass_a(interp, unk, w1at, w1bt, b1.reshape(1, C2))
    scale1, shift1 = _affine(s1, q1, g1, beta1)
    h2, s2, q2 = _pass_b(h1, W2.T, b2.reshape(1, C2), scale1, shift1)
    scale2, shift2 = _affine(s2, q2, g2, beta2)
    out = _pass_c(h2, scale2, shift2)
    return out.reshape(B, N, C2)
